# baseline (speedup 1.0000x reference)
import functools

import jax
import jax.numpy as jnp
from jax import lax
from jax.experimental import pallas as pl
from jax.experimental.pallas import tpu as pltpu
from jax.experimental.pallas import tpu_sc as plsc

N_TOK = 32768
D = 4
B = 16
C = 1024
W = C + 16
PAD = 16
A_MAX = N_TOK - W

_mesh = plsc.VectorSubcoreMesh(core_axis_name="c", subcore_axis_name="s")


@functools.partial(
    pl.kernel,
    mesh=_mesh,
    out_type=jax.ShapeDtypeStruct((D, N_TOK), jnp.float32),
    compiler_params=pltpu.CompilerParams(use_tc_tiling_on_sc=False),
    scratch_types=[
        pltpu.VMEM((17,), jnp.int32),
        *[pltpu.VMEM((C,), jnp.float32) for _ in range(D)],
        *[pltpu.VMEM((W + PAD + 16,), jnp.float32) for _ in range(D)],
        *[pltpu.VMEM((W + PAD + 16,), jnp.float32) for _ in range(D)],
        pltpu.SemaphoreType.DMA,
        pltpu.SemaphoreType.DMA,
        pltpu.SemaphoreType.DMA,
    ],
)
def _normalize_sc(x_hbm, ptr_hbm, mean_hbm, std_hbm, out_hbm,
                  ptr_v, x0, x1, x2, x3, m0, m1, m2, m3, s0, s1, s2, s3,
                  sem_x, sem_m, sem_s):
    xv = [x0, x1, x2, x3]
    mv = [m0, m1, m2, m3]
    sv = [s0, s1, s2, s3]
    wid = lax.axis_index("s") * 2 + lax.axis_index("c")
    c0 = wid * C

    xc = [
        pltpu.async_copy(x_hbm.at[j, pl.ds(c0, C)], xv[j], sem_x)
        for j in range(D)
    ]

    pltpu.sync_copy(ptr_hbm, ptr_v)
    pv = ptr_v[pl.ds(0, 16)]
    starts = [jnp.int32(0)] + [pv[s] for s in range(1, B)]
    ends = starts[1:] + [jnp.int32(N_TOK)]

    for c in xc:
        c.wait()

    lane = lax.iota(jnp.int32, 16)

    for s in range(B):
        start_s = starts[s]
        end_s = ends[s]
        hit = jnp.logical_and(
            jnp.logical_and(start_s < c0 + C, end_s > c0), end_s > start_s
        )

        @pl.when(hit)
        def _():
            d0 = jnp.maximum(start_s - c0, 0)
            dend = jnp.minimum(end_s - c0, C)
            src0 = jnp.maximum(c0 - start_s, 0)
            a = pl.multiple_of(
                jnp.minimum(src0 - lax.rem(src0, jnp.int32(8)), A_MAX), 8
            )
            cm = [
                pltpu.async_copy(mean_hbm.at[j, pl.ds(a, W)],
                                 mv[j].at[pl.ds(PAD, W)], sem_m)
                for j in range(D)
            ]
            cs = [
                pltpu.async_copy(std_hbm.at[j, pl.ds(a, W)],
                                 sv[j].at[pl.ds(PAD, W)], sem_s)
                for j in range(D)
            ]
            for c in cm:
                c.wait()
            for c in cs:
                c.wait()

            woff = c0 - start_s - a + PAD

            def body(g, carry):
                o = g * 16
                pos = lane + o
                mask = jnp.logical_and(pos >= d0, pos < dend)
                off = o + woff
                for j in range(D):
                    xj = xv[j][pl.ds(o, 16)]
                    mj = mv[j][pl.ds(off, 16)]
                    sj = sv[j][pl.ds(off, 16)]
                    yj = (xj - mj) / sj
                    xv[j][pl.ds(o, 16)] = jnp.where(mask, yj, xj)
                return carry

            g0 = lax.shift_right_logical(d0, 4)
            g1 = lax.shift_right_logical(dend - 1, 4) + 1
            lax.fori_loop(g0, g1, body, 0)

    oc = [
        pltpu.async_copy(xv[j], out_hbm.at[j, pl.ds(c0, C)], sem_x)
        for j in range(D)
    ]
    for c in oc:
        c.wait()


def kernel(x, ptr, mean, std):
    return _normalize_sc(x.T, ptr.astype(jnp.int32), mean.T, std.T).T


# speedup vs baseline: 1.0894x; 1.0894x over previous
import functools

import jax
import jax.numpy as jnp
from jax import lax
from jax.experimental import pallas as pl
from jax.experimental.pallas import tpu as pltpu
from jax.experimental.pallas import tpu_sc as plsc

N_TOK = 32768
D = 4
B = 16
C = 1024
TPC = C // 128
NT = N_TOK // 128
W = C + 16
PAD = 16
A_MAX = N_TOK - W
WB = W + PAD + 16

_mesh = plsc.VectorSubcoreMesh(core_axis_name="c", subcore_axis_name="s")


@functools.partial(
    pl.kernel,
    mesh=_mesh,
    out_type=jax.ShapeDtypeStruct((NT, D, 128), jnp.float32),
    compiler_params=pltpu.CompilerParams(use_tc_tiling_on_sc=False),
    scratch_types=[
        pltpu.VMEM((17,), jnp.int32),
        pltpu.VMEM((TPC, D, 128), jnp.float32),
        pltpu.VMEM((D, WB), jnp.float32),
        pltpu.VMEM((D, WB), jnp.float32),
        pltpu.SemaphoreType.DMA,
        pltpu.SemaphoreType.DMA,
        pltpu.SemaphoreType.DMA,
    ],
)
def _normalize_sc(x_hbm, ptr_hbm, mean_hbm, std_hbm, out_hbm,
                  ptr_v, x_v, m_v, s_v, sem_x, sem_m, sem_s):
    wid = lax.axis_index("s") * 2 + lax.axis_index("c")
    c0 = wid * C
    t0 = wid * TPC

    cx = pltpu.async_copy(x_hbm.at[pl.ds(t0, TPC)], x_v, sem_x)

    pltpu.sync_copy(ptr_hbm, ptr_v)
    pv = ptr_v[pl.ds(0, 16)]
    starts = [jnp.int32(0)] + [pv[s] for s in range(1, B)]
    ends = starts[1:] + [jnp.int32(N_TOK)]

    cx.wait()

    lane = lax.iota(jnp.int32, 16)

    for s in range(B):
        start_s = starts[s]
        end_s = ends[s]
        hit = jnp.logical_and(
            jnp.logical_and(start_s < c0 + C, end_s > c0), end_s > start_s
        )

        @pl.when(hit)
        def _():
            d0 = jnp.maximum(start_s - c0, 0)
            dend = jnp.minimum(end_s - c0, C)
            src0 = jnp.maximum(c0 - start_s, 0)
            a = pl.multiple_of(
                jnp.minimum(src0 - lax.rem(src0, jnp.int32(8)), A_MAX), 8
            )
            cm = pltpu.async_copy(mean_hbm.at[:, pl.ds(a, W)],
                                  m_v.at[:, pl.ds(PAD, W)], sem_m)
            cs = pltpu.async_copy(std_hbm.at[:, pl.ds(a, W)],
                                  s_v.at[:, pl.ds(PAD, W)], sem_s)
            cm.wait()
            cs.wait()

            woff = c0 - start_s - a + PAD

            def body(g, carry):
                o = g * 16
                t = lax.shift_right_logical(o, 7)
                p = lax.bitwise_and(o, 127)
                pos = lane + o
                mask = jnp.logical_and(pos >= d0, pos < dend)
                off = o + woff
                for j in range(D):
                    xj = x_v[t, j, pl.ds(p, 16)]
                    mj = m_v[j, pl.ds(off, 16)]
                    sj = s_v[j, pl.ds(off, 16)]
                    yj = (xj - mj) / sj
                    x_v[t, j, pl.ds(p, 16)] = jnp.where(mask, yj, xj)
                return carry

            g0 = lax.shift_right_logical(d0, 4)
            g1 = lax.shift_right_logical(dend - 1, 4) + 1
            lax.fori_loop(g0, g1, body, 0)

    pltpu.async_copy(x_v, out_hbm.at[pl.ds(t0, TPC)], sem_x).wait()


def kernel(x, ptr, mean, std):
    xr = x.reshape(NT, 128, D).swapaxes(1, 2)
    yr = _normalize_sc(xr, ptr.astype(jnp.int32), mean.T, std.T)
    return yr.swapaxes(1, 2).reshape(N_TOK, D)


# speedup vs baseline: 1.1003x; 1.0100x over previous
import functools

import jax
import jax.numpy as jnp
from jax import lax
from jax.experimental import pallas as pl
from jax.experimental.pallas import tpu as pltpu
from jax.experimental.pallas import tpu_sc as plsc

N_TOK = 32768
D = 4
B = 16
C = 1024
TPC = C // 128
NT = N_TOK // 128
WT = TPC + 1
TA_MAX = NT - WT
PAD = 16
WB = PAD + WT * 128 + 16

_mesh = plsc.VectorSubcoreMesh(core_axis_name="c", subcore_axis_name="s")


@functools.partial(
    pl.kernel,
    mesh=_mesh,
    out_type=jax.ShapeDtypeStruct((NT, D, 128), jnp.float32),
    compiler_params=pltpu.CompilerParams(use_tc_tiling_on_sc=False),
    scratch_types=[
        pltpu.VMEM((33,), jnp.int32),
        pltpu.VMEM((TPC, D, 128), jnp.float32),
        pltpu.VMEM((WT, D, 128), jnp.float32),
        pltpu.VMEM((WT, D, 128), jnp.float32),
        *[pltpu.VMEM((WB,), jnp.float32) for _ in range(D)],
        *[pltpu.VMEM((WB,), jnp.float32) for _ in range(D)],
        pltpu.SemaphoreType.DMA,
        pltpu.SemaphoreType.DMA,
        pltpu.SemaphoreType.DMA,
    ],
)
def _normalize_sc(x_hbm, ptr_hbm, mean_hbm, std_hbm, out_hbm,
                  ptr_v, x_v, m3, s3, m0, m1, m2, m3v, s0, s1, s2, s3v,
                  sem_x, sem_m, sem_s):
    mv = [m0, m1, m2, m3v]
    sv = [s0, s1, s2, s3v]
    wid = lax.axis_index("s") * 2 + lax.axis_index("c")
    c0 = wid * C
    t0 = wid * TPC

    cx = pltpu.async_copy(x_hbm.at[pl.ds(t0, TPC)], x_v, sem_x)

    pltpu.sync_copy(ptr_hbm, ptr_v.at[pl.ds(0, 17)])

    cx.wait()

    lane = lax.iota(jnp.int32, 16)

    def seg_body(s, carry):
        start_s = ptr_v[pl.ds(s, 16)][0]
        end_s = ptr_v[pl.ds(s + 1, 16)][0]
        hit = jnp.logical_and(
            jnp.logical_and(start_s < c0 + C, end_s > c0), end_s > start_s
        )

        @pl.when(hit)
        def _():
            d0 = jnp.maximum(start_s - c0, 0)
            dend = jnp.minimum(end_s - c0, C)
            src0 = jnp.maximum(c0 - start_s, 0)
            ta = jnp.minimum(lax.shift_right_logical(src0, 7), TA_MAX)
            cm = pltpu.async_copy(mean_hbm.at[pl.ds(ta, WT)], m3, sem_m)
            cs = pltpu.async_copy(std_hbm.at[pl.ds(ta, WT)], s3, sem_s)
            cm.wait()
            cs.wait()

            def rearrange(t, carry):
                for j in range(D):
                    for k in range(128 // 16):
                        o16 = k * 16
                        mv[j][pl.ds(PAD + t * 128 + o16, 16)] = (
                            m3[t, j, pl.ds(o16, 16)])
                        sv[j][pl.ds(PAD + t * 128 + o16, 16)] = (
                            s3[t, j, pl.ds(o16, 16)])
                return carry

            lax.fori_loop(0, WT, rearrange, 0)

            woff = c0 - start_s - ta * 128 + PAD

            def body(g, carry):
                o = g * 16
                t = lax.shift_right_logical(o, 7)
                p = lax.bitwise_and(o, 127)
                pos = lane + o
                mask = jnp.logical_and(pos >= d0, pos < dend)
                off = o + woff
                for j in range(D):
                    xj = x_v[t, j, pl.ds(p, 16)]
                    mj = mv[j][pl.ds(off, 16)]
                    sj = sv[j][pl.ds(off, 16)]
                    yj = (xj - mj) / sj
                    x_v[t, j, pl.ds(p, 16)] = jnp.where(mask, yj, xj)
                return carry

            g0 = lax.shift_right_logical(d0, 4)
            g1 = lax.shift_right_logical(dend - 1, 4) + 1
            lax.fori_loop(g0, g1, body, 0)

        return carry

    lax.fori_loop(0, B, seg_body, 0)

    pltpu.async_copy(x_v, out_hbm.at[pl.ds(t0, TPC)], sem_x).wait()


def kernel(x, ptr, mean, std):
    xr = x.reshape(NT, 128, D).swapaxes(1, 2)
    mr = mean.reshape(NT, 128, D).swapaxes(1, 2)
    sr = std.reshape(NT, 128, D).swapaxes(1, 2)
    yr = _normalize_sc(xr, ptr.astype(jnp.int32), mr, sr)
    return yr.swapaxes(1, 2).reshape(N_TOK, D)
